# Initial kernel scaffold; baseline (speedup 1.0000x reference)
import jax
import jax.numpy as jnp
from jax.experimental import pallas as pl


def kernel(hidden_states, ln1, ln2, Wq, Wk, Wv, Wo, Wr, Wg, Wu, Wd):
    raise NotImplementedError("write your pallas kernel here")


# speedup vs baseline: 1.1243x; 1.1243x over previous
import functools

import jax
import jax.numpy as jnp
import numpy as np
from jax.experimental import pallas as pl
from jax.experimental.pallas import tpu as pltpu

B, S, H = 1, 2048, 1024
NH, HD = 16, 64
E, K, I = 8, 2, 512
EPS = 1e-6
BS = 512

_PREC = jax.lax.Precision.DEFAULT


def _dot_t(a, b):
    return jax.lax.dot_general(a, b, (((1,), (1,)), ((), ())),
                               preferred_element_type=jnp.float32,
                               precision=_PREC)


def _dot(a, b):
    return jax.lax.dot_general(a, b, (((1,), (0,)), ((), ())),
                               preferred_element_type=jnp.float32,
                               precision=_PREC)


def _rms(x, w):
    v = jnp.mean(x * x, axis=-1, keepdims=True)
    return x * jax.lax.rsqrt(v + EPS) * w


def _qkv_kernel(x_ref, ln_ref, wq_ref, wk_ref, wv_ref, q_ref, k_ref, v_ref):
    r = _rms(x_ref[...], ln_ref[...])
    q_ref[...] = _dot_t(r, wq_ref[...])
    k_ref[...] = _dot_t(r, wk_ref[...])
    v_ref[...] = _dot_t(r, wv_ref[...])


def _attn_kernel(q_ref, k_ref, v_ref, o_ref):
    s = _dot_t(q_ref[0], k_ref[0]) * (1.0 / np.sqrt(HD))
    m = jnp.max(s, axis=-1, keepdims=True)
    p = jnp.exp(s - m)
    p = p / jnp.sum(p, axis=-1, keepdims=True)
    o_ref[0] = _dot(p, v_ref[0])


def _post_kernel(x_ref, o_ref, wo_ref, ln_ref, wr_ref,
                 x1_ref, r2_ref, comb_ref):
    x1 = x_ref[...] + _dot_t(o_ref[...], wo_ref[...])
    x1_ref[...] = x1
    r2 = _rms(x1, ln_ref[...])
    r2_ref[...] = r2
    logits = _dot_t(r2, wr_ref[...])
    lane = jax.lax.broadcasted_iota(jnp.int32, (BS, 128), 1)
    neg = jnp.where(lane < E, logits, -1e30)
    t1v = jnp.max(neg, axis=-1, keepdims=True)
    i1 = jnp.min(jnp.where(neg == t1v, lane, 999), axis=-1, keepdims=True)
    m0 = lane == i1
    neg2 = jnp.where(m0, -1e30, neg)
    t2v = jnp.max(neg2, axis=-1, keepdims=True)
    i2 = jnp.min(jnp.where(neg2 == t2v, lane, 999), axis=-1, keepdims=True)
    m1 = lane == i2
    z = jnp.exp(t2v - t1v)
    w0 = 1.0 / (1.0 + z)
    w1 = z / (1.0 + z)
    comb_ref[...] = jnp.where(m0, w0, 0.0) + jnp.where(m1, w1, 0.0)


def _moe_dense_kernel(r2_ref, comb_ref, x1_ref, wg_ref, wu_ref, wd_ref,
                      out_ref):
    e = pl.program_id(1)
    r2 = r2_ref[...]
    g = _dot_t(r2, wg_ref[0])
    u = _dot_t(r2, wu_ref[0])
    h = g * jax.nn.sigmoid(g) * u
    y = _dot_t(h, wd_ref[0])
    lane = jax.lax.broadcasted_iota(jnp.int32, (BS, 128), 1)
    ce = jnp.sum(jnp.where(lane == e, comb_ref[...], 0.0), axis=-1,
                 keepdims=True)

    @pl.when(e == 0)
    def _():
        out_ref[...] = x1_ref[...] + ce * y

    @pl.when(e != 0)
    def _():
        out_ref[...] = out_ref[...] + ce * y


def kernel(hidden_states, ln1, ln2, Wq, Wk, Wv, Wo, Wr, Wg, Wu, Wd):
    x = hidden_states.reshape(S, H)
    ln1r = ln1.reshape(1, H)
    ln2r = ln2.reshape(1, H)
    wr_pad = jnp.zeros((128, H), jnp.float32).at[:E].set(Wr)

    full = lambda shape: pl.BlockSpec(shape, lambda i: (0,) * len(shape))
    rowblk = pl.BlockSpec((BS, H), lambda i: (i, 0))

    q, k, v = pl.pallas_call(
        _qkv_kernel,
        grid=(S // BS,),
        in_specs=[rowblk, full((1, H)), full((H, H)), full((H, H)),
                  full((H, H))],
        out_specs=[rowblk, rowblk, rowblk],
        out_shape=[jax.ShapeDtypeStruct((S, H), jnp.float32)] * 3,
    )(x, ln1r, Wq, Wk, Wv)

    tohead = lambda a: a.reshape(S, NH, HD).transpose(1, 0, 2)
    qh, kh, vh = tohead(q), tohead(k), tohead(v)
    oh = pl.pallas_call(
        _attn_kernel,
        grid=(NH, S // BS),
        in_specs=[
            pl.BlockSpec((1, BS, HD), lambda h, i: (h, i, 0)),
            pl.BlockSpec((1, S, HD), lambda h, i: (h, 0, 0)),
            pl.BlockSpec((1, S, HD), lambda h, i: (h, 0, 0)),
        ],
        out_specs=pl.BlockSpec((1, BS, HD), lambda h, i: (h, i, 0)),
        out_shape=jax.ShapeDtypeStruct((NH, S, HD), jnp.float32),
    )(qh, kh, vh)
    o = oh.transpose(1, 0, 2).reshape(S, H)

    x1, r2, comb = pl.pallas_call(
        _post_kernel,
        grid=(S // BS,),
        in_specs=[rowblk, rowblk, full((H, H)), full((1, H)),
                  full((128, H))],
        out_specs=[rowblk, rowblk, pl.BlockSpec((BS, 128), lambda i: (i, 0))],
        out_shape=[
            jax.ShapeDtypeStruct((S, H), jnp.float32),
            jax.ShapeDtypeStruct((S, H), jnp.float32),
            jax.ShapeDtypeStruct((S, 128), jnp.float32),
        ],
    )(x, o, Wo, ln2r, wr_pad)

    out = pl.pallas_call(
        _moe_dense_kernel,
        grid=(S // BS, E),
        in_specs=[
            pl.BlockSpec((BS, H), lambda i, e: (i, 0)),
            pl.BlockSpec((BS, 128), lambda i, e: (i, 0)),
            pl.BlockSpec((BS, H), lambda i, e: (i, 0)),
            pl.BlockSpec((1, I, H), lambda i, e: (e, 0, 0)),
            pl.BlockSpec((1, I, H), lambda i, e: (e, 0, 0)),
            pl.BlockSpec((1, H, I), lambda i, e: (e, 0, 0)),
        ],
        out_specs=pl.BlockSpec((BS, H), lambda i, e: (i, 0)),
        out_shape=jax.ShapeDtypeStruct((S, H), jnp.float32),
    )(r2, comb, x1, Wg, Wu, Wd)

    return out.reshape(B, S, H)
